# Initial kernel scaffold; baseline (speedup 1.0000x reference)
import jax
import jax.numpy as jnp
from jax.experimental import pallas as pl


def kernel(input, index):
    raise NotImplementedError("write your pallas kernel here")


# speedup vs baseline: 1.2830x; 1.2830x over previous
import jax
import jax.numpy as jnp
from jax import lax
from jax.experimental import pallas as pl
from jax.experimental.pallas import tpu as pltpu
from jax.experimental.pallas import tpu_sc as plsc

_D = 32
_L = 128
_K = 8
_NC = 2
_NS = 16
_NW = _NC * _NS


def _gather_body(table_hbm, idx_hbm, out_hbm, idx_v, rows_v, sem):
    wid = lax.axis_index("s") * _NC + lax.axis_index("c")
    n_vecs = idx_hbm.shape[0]
    per_w = n_vecs // _NW
    n_chunks = per_w // _K
    base = wid * per_w

    def chunk(c, carry):
        r0 = base + c * _K
        pltpu.sync_copy(idx_hbm.at[pl.ds(r0, _K)], idx_v)
        copies = [
            pltpu.async_copy(table_hbm.at[idx_v.at[j]], rows_v.at[j], sem)
            for j in range(_K)
        ]
        for cp in copies:
            cp.wait()
        pltpu.sync_copy(rows_v, out_hbm.at[pl.ds(r0, _K)])
        return carry

    lax.fori_loop(0, n_chunks, chunk, 0)


@jax.jit
def kernel(input, index):
    idx_flat = index.reshape(-1).astype(jnp.int32)
    n = idx_flat.shape[0]
    idx2d = idx_flat.reshape(n // _L, _L)
    mesh = plsc.VectorSubcoreMesh(core_axis_name="c", subcore_axis_name="s")
    out = pl.kernel(
        _gather_body,
        mesh=mesh,
        out_type=jax.ShapeDtypeStruct((n // _L, _L, _D), jnp.float32),
        scratch_types=[
            pltpu.VMEM((_K, _L), jnp.int32),
            pltpu.VMEM((_K, _L, _D), jnp.float32),
            pltpu.SemaphoreType.DMA,
        ],
        compiler_params=pltpu.CompilerParams(use_tc_tiling_on_sc=False),
    )(input, idx2d)
    return out.reshape(index.shape + (_D,))


# speedup vs baseline: 1.3098x; 1.0209x over previous
import jax
import jax.numpy as jnp
from jax import lax
from jax.experimental import pallas as pl
from jax.experimental.pallas import tpu as pltpu
from jax.experimental.pallas import tpu_sc as plsc

_D = 32
_L = 128
_K = 10
_NC = 2
_NS = 16
_NW = _NC * _NS


def _gather_body(table_hbm, idx_hbm, out_hbm, idx_all, rows0, rows1,
                 gsem0, gsem1, osem):
    wid = lax.axis_index("s") * _NC + lax.axis_index("c")
    n_vecs = idx_hbm.shape[0]
    per_w = n_vecs // _NW
    n_chunks = per_w // _K
    base = wid * per_w

    pltpu.sync_copy(idx_hbm.at[pl.ds(base, per_w)], idx_all)

    rows = (rows0, rows1)
    gsem = (gsem0, gsem1)

    def fire_gather(c, b):
        for j in range(_K):
            pltpu.async_copy(
                table_hbm.at[idx_all.at[c * _K + j]], rows[b].at[j], gsem[b])

    def drain_gather(c, b):
        for j in range(_K):
            pltpu.make_async_copy(
                table_hbm.at[idx_all.at[c * _K + j]], rows[b].at[j],
                gsem[b]).wait()

    def fire_store(c, b):
        pltpu.async_copy(
            rows[b], out_hbm.at[pl.ds(base + c * _K, _K)], osem)

    def drain_store(c, b):
        pltpu.make_async_copy(
            rows[b], out_hbm.at[pl.ds(base + c * _K, _K)], osem).wait()

    fire_gather(0, 0)
    fire_gather(1, 1)

    def pair(g, carry):
        for b in range(2):
            c = 2 * g + b
            drain_gather(c, b)
            fire_store(c, b)
            drain_store(c, b)
            fire_gather(c + 2, b)
        return carry

    lax.fori_loop(0, n_chunks // 2 - 1, pair, 0)

    for b in range(2):
        c = n_chunks - 2 + b
        drain_gather(c, b)
        fire_store(c, b)
        drain_store(c, b)


@jax.jit
def kernel(input, index):
    idx_flat = index.reshape(-1).astype(jnp.int32)
    n = idx_flat.shape[0]
    idx2d = idx_flat.reshape(n // _L, _L)
    mesh = plsc.VectorSubcoreMesh(core_axis_name="c", subcore_axis_name="s")
    per_w = (n // _L) // _NW
    out = pl.kernel(
        _gather_body,
        mesh=mesh,
        out_type=jax.ShapeDtypeStruct((n // _L, _L, _D), jnp.float32),
        scratch_types=[
            pltpu.VMEM((per_w, _L), jnp.int32),
            pltpu.VMEM((_K, _L, _D), jnp.float32),
            pltpu.VMEM((_K, _L, _D), jnp.float32),
            pltpu.SemaphoreType.DMA,
            pltpu.SemaphoreType.DMA,
            pltpu.SemaphoreType.DMA,
        ],
        compiler_params=pltpu.CompilerParams(use_tc_tiling_on_sc=False),
    )(input, idx2d)
    return out.reshape(index.shape + (_D,))
